# baseline (speedup 1.0000x reference)
import functools

import jax
import jax.numpy as jnp
from jax import lax
from jax.experimental import pallas as pl
from jax.experimental.pallas import tpu as pltpu
from jax.experimental.pallas import tpu_sc as plsc

B = 4096
D_DENSE = 13
N_SPARSE = 100
EMB = 128
VOCAB = 101
JROWS = 104
VCOLS = 128
NW = 32
ROWS_PER_W = B // NW
GROUPS = ROWS_PER_W // 16
UNROLL = 4


def _tc_body(emb_ref, ws_ref, den_ref, wd_ref, b_ref, bias_ref, v_ref, p_ref):
    emb = emb_ref[...]
    row = lax.broadcasted_iota(jnp.int32, (VOCAB, EMB), 0)
    emb = jnp.where(row == 0, 0.0, emb)
    v_ref[:N_SPARSE, :VOCAB] = lax.dot_general(
        ws_ref[...], emb, (((1,), (1,)), ((), ())),
        precision=lax.Precision.HIGHEST,
        preferred_element_type=jnp.float32)
    c = b_ref[0, 0] + bias_ref[0, 0]
    p = lax.dot_general(
        wd_ref[...], den_ref[...], (((1,), (0,)), ((), ())),
        preferred_element_type=jnp.float32)
    p_ref[...] = p + c


_tc_call = pl.pallas_call(
    _tc_body,
    out_shape=[
        jax.ShapeDtypeStruct((JROWS, VCOLS), jnp.float32),
        jax.ShapeDtypeStruct((1, B), jnp.float32),
    ],
    in_specs=[
        pl.BlockSpec(memory_space=pltpu.VMEM),
        pl.BlockSpec(memory_space=pltpu.VMEM),
        pl.BlockSpec(memory_space=pltpu.VMEM),
        pl.BlockSpec(memory_space=pltpu.VMEM),
        pl.BlockSpec(memory_space=pltpu.SMEM),
        pl.BlockSpec(memory_space=pltpu.SMEM),
    ],
)

_mesh = plsc.VectorSubcoreMesh(
    core_axis_name="c", subcore_axis_name="s", num_cores=2, num_subcores=16)


@functools.partial(
    pl.kernel,
    out_type=jax.ShapeDtypeStruct((B,), jnp.float32),
    mesh=_mesh,
    scratch_types=[
        pltpu.VMEM((JROWS, VCOLS), jnp.float32),
        pltpu.VMEM((N_SPARSE, ROWS_PER_W), jnp.int32),
        pltpu.VMEM((ROWS_PER_W,), jnp.float32),
        pltpu.VMEM((ROWS_PER_W,), jnp.float32),
        pltpu.SemaphoreType.DMA,
    ],
    compiler_params=pltpu.CompilerParams(needs_layout_passes=False),
)
def _sc_kernel(v_hbm, idx_hbm, p_hbm, out_hbm, v_v, idx_v, p_v, o_v, sem):
    wid = lax.axis_index("s") * 2 + lax.axis_index("c")
    base = wid * ROWS_PER_W
    c1 = pltpu.async_copy(v_hbm, v_v, sem)
    c2 = pltpu.async_copy(idx_hbm.at[:, pl.ds(base, ROWS_PER_W)], idx_v, sem)
    c3 = pltpu.async_copy(p_hbm.at[pl.ds(base, ROWS_PER_W)], p_v, sem)
    c1.wait()
    c2.wait()
    c3.wait()
    lanes = lax.iota(jnp.int32, 16)
    mlo = lanes < 8
    mhi = lanes >= 8
    for g in range(GROUPS):
        cols = lanes + g * 16

        def body(jj, acc, cols=cols):
            j0 = jj * UNROLL
            vals = []
            for u in range(UNROLL):
                jv = jnp.full((16,), j0 + u, jnp.int32)
                voc = plsc.load_gather(idx_v, [jv, cols])
                vlo = plsc.load_gather(v_v, [jv, voc], mask=mlo)
                vhi = plsc.load_gather(v_v, [jv, voc], mask=mhi)
                vals.append(jnp.where(mlo, vlo, vhi))
            return acc + ((vals[0] + vals[1]) + (vals[2] + vals[3]))

        acc = lax.fori_loop(0, N_SPARSE // UNROLL, body,
                            jnp.zeros((16,), jnp.float32))
        acc = acc + p_v[pl.ds(g * 16, 16)]
        o_v[pl.ds(g * 16, 16)] = 1.0 / (1.0 + jnp.exp(-acc))
    pltpu.sync_copy(o_v, out_hbm.at[pl.ds(base, ROWS_PER_W)])


def kernel(dense_features, sparse_features, emb_table, W, b, bias):
    idx_t = sparse_features.astype(jnp.int32).T
    den_t = dense_features.T
    wd = W[:, :D_DENSE]
    ws = W[0, D_DENSE:].reshape(N_SPARSE, EMB)
    v, p = _tc_call(emb_table, ws, den_t, wd,
                    b.reshape(1, 1), bias.reshape(1, 1))
    return _sc_kernel(v, idx_t, p.reshape(B))


# speedup vs baseline: 1.0145x; 1.0145x over previous
import functools

import jax
import jax.numpy as jnp
from jax import lax
from jax.experimental import pallas as pl
from jax.experimental.pallas import tpu as pltpu
from jax.experimental.pallas import tpu_sc as plsc

B = 4096
D_DENSE = 13
N_SPARSE = 100
EMB = 128
VOCAB = 101
JROWS = 104
VCOLS = 128
NW = 32
ROWS_PER_W = B // NW
GROUPS = ROWS_PER_W // 16
UNROLL = 4


def _tc_body(emb_ref, ws_ref, den_ref, wd_ref, b_ref, bias_ref, v_ref, p_ref):
    emb = emb_ref[...]
    row = lax.broadcasted_iota(jnp.int32, (VOCAB, EMB), 0)
    emb = jnp.where(row == 0, 0.0, emb)
    v_ref[:N_SPARSE, :VOCAB] = lax.dot_general(
        ws_ref[...], emb, (((1,), (1,)), ((), ())),
        precision=lax.Precision.HIGHEST,
        preferred_element_type=jnp.float32)
    c = b_ref[0, 0] + bias_ref[0, 0]
    p = lax.dot_general(
        wd_ref[...], den_ref[...], (((1,), (0,)), ((), ())),
        preferred_element_type=jnp.float32)
    p_ref[...] = p + c


_tc_call = pl.pallas_call(
    _tc_body,
    out_shape=[
        jax.ShapeDtypeStruct((JROWS, VCOLS), jnp.float32),
        jax.ShapeDtypeStruct((1, B), jnp.float32),
    ],
    in_specs=[
        pl.BlockSpec(memory_space=pltpu.VMEM),
        pl.BlockSpec(memory_space=pltpu.VMEM),
        pl.BlockSpec(memory_space=pltpu.VMEM),
        pl.BlockSpec(memory_space=pltpu.VMEM),
        pl.BlockSpec(memory_space=pltpu.SMEM),
        pl.BlockSpec(memory_space=pltpu.SMEM),
    ],
)

_mesh = plsc.VectorSubcoreMesh(
    core_axis_name="c", subcore_axis_name="s", num_cores=2, num_subcores=16)


@functools.partial(
    pl.kernel,
    out_type=jax.ShapeDtypeStruct((B,), jnp.float32),
    mesh=_mesh,
    scratch_types=[
        pltpu.VMEM((JROWS, VCOLS), jnp.float32),
        pltpu.VMEM((N_SPARSE, ROWS_PER_W), jnp.int32),
        pltpu.VMEM((ROWS_PER_W,), jnp.float32),
        pltpu.VMEM((ROWS_PER_W,), jnp.float32),
        pltpu.SemaphoreType.DMA,
    ],
    compiler_params=pltpu.CompilerParams(needs_layout_passes=False),
)
def _sc_kernel(v_hbm, idx_hbm, p_hbm, out_hbm, v_v, idx_v, p_v, o_v, sem):
    wid = lax.axis_index("s") * 2 + lax.axis_index("c")
    base = wid * ROWS_PER_W
    c1 = pltpu.async_copy(v_hbm, v_v, sem)
    c2 = pltpu.async_copy(idx_hbm.at[:, pl.ds(base, ROWS_PER_W)], idx_v, sem)
    c3 = pltpu.async_copy(p_hbm.at[pl.ds(base, ROWS_PER_W)], p_v, sem)
    c1.wait()
    c2.wait()
    c3.wait()
    lanes = lax.iota(jnp.int32, 16)
    for g in range(GROUPS):
        cols = lanes + g * 16

        def body(jj, acc, cols=cols):
            j0 = jj * UNROLL
            vals = []
            for u in range(UNROLL):
                jv = jnp.full((16,), j0 + u, jnp.int32)
                voc = plsc.load_gather(idx_v, [jv, cols])
                vals.append(plsc.load_gather(v_v, [jv, voc]))
            return acc + ((vals[0] + vals[1]) + (vals[2] + vals[3]))

        acc = lax.fori_loop(0, N_SPARSE // UNROLL, body,
                            jnp.zeros((16,), jnp.float32))
        acc = acc + p_v[pl.ds(g * 16, 16)]
        o_v[pl.ds(g * 16, 16)] = 1.0 / (1.0 + jnp.exp(-acc))
    pltpu.sync_copy(o_v, out_hbm.at[pl.ds(base, ROWS_PER_W)])


def kernel(dense_features, sparse_features, emb_table, W, b, bias):
    idx_t = sparse_features.astype(jnp.int32).T
    den_t = dense_features.T
    wd = W[:, :D_DENSE]
    ws = W[0, D_DENSE:].reshape(N_SPARSE, EMB)
    v, p = _tc_call(emb_table, ws, den_t, wd,
                    b.reshape(1, 1), bias.reshape(1, 1))
    return _sc_kernel(v, idx_t, p.reshape(B))
